# Initial kernel scaffold; baseline (speedup 1.0000x reference)
import jax
import jax.numpy as jnp
from jax.experimental import pallas as pl


def kernel(x, edge_index, win_w, win_b, w1, w2, wout_w, wout_b):
    raise NotImplementedError("write your pallas kernel here")


# speedup vs baseline: 9.1867x; 9.1867x over previous
import functools
import math

import jax
import jax.numpy as jnp
from jax import lax
from jax.experimental import pallas as pl
from jax.experimental.pallas import tpu as pltpu
from jax.experimental.pallas import tpu_sc as plsc

N = 10000
D = 128
E = 320000
NLAYERS = 8
NCLS = 40
ALPHA = 0.1
LAMDA = 0.5

N_PAD = 10240
NC, NS = 2, 16
NW = NC * NS
CH = 128
J = 79
EPW = J * CH
E_PAD = NW * EPW
RPT = N_PAD // NS
BLK = 1024
DEGW = 16

BETA = [float(math.log(LAMDA / (i + 1) + 1.0)) for i in range(NLAYERS)]
C1 = [(1.0 - b) * (1.0 - ALPHA) for b in BETA]
C2 = [(1.0 - b) * ALPHA for b in BETA]

_MESH = plsc.VectorSubcoreMesh(core_axis_name="c", subcore_axis_name="s")


def _zero_zrow(zrow_v, width):
    z16 = jnp.zeros((16,), jnp.float32)
    for r in range(16):
        for c in range(width // 16):
            zrow_v[r, pl.ds(c * 16, 16)] = z16


def _sc_degree_body(dst_hbm, deg_out, dst_v, ones_v, zrow_v, degT):
    cid = lax.axis_index("c")
    sid = lax.axis_index("s")
    wid = cid * NS + sid
    e0 = jnp.where(lax.iota(jnp.int32, 16) == 0, 1.0, 0.0)
    z16 = jnp.zeros((16,), jnp.float32)

    def fill(r, carry):
        ones_v[r, pl.ds(0, 16)] = e0
        for c in range(1, D // 16):
            ones_v[r, pl.ds(c * 16, 16)] = z16
        return carry
    lax.fori_loop(0, CH, fill, 0)
    _zero_zrow(zrow_v, D)

    def zb(j, carry):
        pltpu.sync_copy(zrow_v, degT.at[pl.ds(sid * RPT + j * 16, 16), :])
        return carry
    lax.fori_loop(0, RPT // 16, zb, 0)
    plsc.subcore_barrier()

    pltpu.sync_copy(dst_hbm.at[wid], dst_v)

    def body(j, carry):
        pltpu.sync_copy(ones_v, degT.at[dst_v.at[j]], add=True)
        return carry
    lax.fori_loop(0, J, body, 0)
    plsc.subcore_barrier()
    pltpu.sync_copy(degT.at[pl.ds(sid * RPT, RPT), :],
                    deg_out.at[cid, pl.ds(sid * RPT, RPT), :])


_sc_degree = pl.kernel(
    _sc_degree_body,
    out_type=jax.ShapeDtypeStruct((NC, N_PAD, D), jnp.float32),
    mesh=_MESH,
    scratch_types=[
        pltpu.VMEM((J, CH), jnp.int32),
        pltpu.VMEM((CH, D), jnp.float32),
        pltpu.VMEM((16, D), jnp.float32),
        pltpu.VMEM_SHARED((N_PAD, D), jnp.float32),
    ],
)


def _sc_prop_body(u_hbm, src_hbm, dst_hbm, s_out, src_v, dst_v, rows_v,
                  zrow_v, g_sh):
    cid = lax.axis_index("c")
    sid = lax.axis_index("s")
    wid = cid * NS + sid
    _zero_zrow(zrow_v, D)

    def zb(j, carry):
        pltpu.sync_copy(zrow_v, g_sh.at[pl.ds(sid * RPT + j * 16, 16), :])
        return carry
    lax.fori_loop(0, RPT // 16, zb, 0)
    plsc.subcore_barrier()

    pltpu.sync_copy(src_hbm.at[wid], src_v)
    pltpu.sync_copy(dst_hbm.at[wid], dst_v)

    def body(j, carry):
        pltpu.sync_copy(u_hbm.at[src_v.at[j]], rows_v)
        pltpu.sync_copy(rows_v, g_sh.at[dst_v.at[j]], add=True)
        return carry
    lax.fori_loop(0, J, body, 0)
    plsc.subcore_barrier()
    pltpu.sync_copy(g_sh.at[pl.ds(sid * RPT, RPT), :],
                    s_out.at[cid, pl.ds(sid * RPT, RPT), :])


_sc_prop = pl.kernel(
    _sc_prop_body,
    out_type=jax.ShapeDtypeStruct((NC, N_PAD, D), jnp.float32),
    mesh=_MESH,
    scratch_types=[
        pltpu.VMEM((J, CH), jnp.int32),
        pltpu.VMEM((J, CH), jnp.int32),
        pltpu.VMEM((CH, D), jnp.float32),
        pltpu.VMEM((16, D), jnp.float32),
        pltpu.VMEM_SHARED((N_PAD, D), jnp.float32),
    ],
)


def _tc_prelude_body(x_ref, winw_ref, winb_ref, w2_ref, degT_ref,
                     u_ref, dinv_ref, z2_ref):
    h0 = jnp.maximum(x_ref[...] @ winw_ref[...] + winb_ref[...], 0.0)
    deg = jnp.sum(degT_ref[0] + degT_ref[1], axis=1) + 1.0
    dinv2 = jnp.broadcast_to(lax.rsqrt(deg)[:, None], (BLK, D))
    u_ref[...] = dinv2 * h0
    dinv_ref[...] = dinv2
    for i in range(NLAYERS):
        z2_ref[i] = C2[i] * h0 + BETA[i] * (h0 @ w2_ref[i])


_tc_prelude = pl.pallas_call(
    _tc_prelude_body,
    grid=(N_PAD // BLK,),
    in_specs=[
        pl.BlockSpec((BLK, D), lambda b: (b, 0)),
        pl.BlockSpec((D, D), lambda b: (0, 0)),
        pl.BlockSpec((1, D), lambda b: (0, 0)),
        pl.BlockSpec((NLAYERS, D, D), lambda b: (0, 0, 0)),
        pl.BlockSpec((NC, BLK, D), lambda b: (0, b, 0)),
    ],
    out_specs=[
        pl.BlockSpec((BLK, D), lambda b: (b, 0)),
        pl.BlockSpec((BLK, D), lambda b: (b, 0)),
        pl.BlockSpec((NLAYERS, BLK, D), lambda b: (0, b, 0)),
    ],
    out_shape=[
        jax.ShapeDtypeStruct((N_PAD, D), jnp.float32),
        jax.ShapeDtypeStruct((N_PAD, D), jnp.float32),
        jax.ShapeDtypeStruct((NLAYERS, N_PAD, D), jnp.float32),
    ],
)


def _tc_layer_body(i, s_ref, u_ref, dinv_ref, z2_ref, w1_ref, unew_ref):
    t = s_ref[0] + s_ref[1] + u_ref[...]
    g = dinv_ref[...] * t
    h = jnp.maximum(C1[i] * g + BETA[i] * (g @ w1_ref[0]) + z2_ref[0], 0.0)
    unew_ref[...] = dinv_ref[...] * h


def _make_tc_layer(i):
    return pl.pallas_call(
        functools.partial(_tc_layer_body, i),
        grid=(N_PAD // BLK,),
        in_specs=[
            pl.BlockSpec((NC, BLK, D), lambda b: (0, b, 0)),
            pl.BlockSpec((BLK, D), lambda b: (b, 0)),
            pl.BlockSpec((BLK, D), lambda b: (b, 0)),
            pl.BlockSpec((1, BLK, D), lambda b, i=i: (i, b, 0)),
            pl.BlockSpec((1, D, D), lambda b, i=i: (i, 0, 0)),
        ],
        out_specs=pl.BlockSpec((BLK, D), lambda b: (b, 0)),
        out_shape=jax.ShapeDtypeStruct((N_PAD, D), jnp.float32),
    )


_tc_layers = [_make_tc_layer(i) for i in range(NLAYERS - 1)]


def _tc_last_body(s_ref, u_ref, dinv_ref, z2_ref, w1_ref, woutw_ref,
                  woutb_ref, out_ref):
    i = NLAYERS - 1
    t = s_ref[0] + s_ref[1] + u_ref[...]
    g = dinv_ref[...] * t
    h = jnp.maximum(C1[i] * g + BETA[i] * (g @ w1_ref[0]) + z2_ref[0], 0.0)
    out_ref[...] = h @ woutw_ref[...] + woutb_ref[...]


_tc_last = pl.pallas_call(
    _tc_last_body,
    grid=(N_PAD // BLK,),
    in_specs=[
        pl.BlockSpec((NC, BLK, D), lambda b: (0, b, 0)),
        pl.BlockSpec((BLK, D), lambda b: (b, 0)),
        pl.BlockSpec((BLK, D), lambda b: (b, 0)),
        pl.BlockSpec((1, BLK, D), lambda b: (NLAYERS - 1, b, 0)),
        pl.BlockSpec((1, D, D), lambda b: (NLAYERS - 1, 0, 0)),
        pl.BlockSpec((D, D), lambda b: (0, 0)),
        pl.BlockSpec((1, D), lambda b: (0, 0)),
    ],
    out_specs=pl.BlockSpec((BLK, D), lambda b: (b, 0)),
    out_shape=jax.ShapeDtypeStruct((N_PAD, D), jnp.float32),
)


def kernel(x, edge_index, win_w, win_b, w1, w2, wout_w, wout_b):
    src = edge_index[0].astype(jnp.int32)
    dst = edge_index[1].astype(jnp.int32)
    pad_idx = jnp.full((E_PAD - E,), N, jnp.int32)
    srcp = jnp.concatenate([src, pad_idx]).reshape(NW, J, CH)
    dstp = jnp.concatenate([dst, pad_idx]).reshape(NW, J, CH)
    xp = jnp.pad(x, ((0, N_PAD - N), (0, 0)))
    woutwp = jnp.pad(wout_w, ((0, 0), (0, D - NCLS)))
    woutbp = jnp.pad(wout_b, (0, D - NCLS)).reshape(1, D)
    winb2 = win_b.reshape(1, D)

    degT = _sc_degree(dstp)
    u, dinv2, z2 = _tc_prelude(xp, win_w, winb2, w2, degT)
    for i in range(NLAYERS - 1):
        s = _sc_prop(u, srcp, dstp)
        u = _tc_layers[i](s, u, dinv2, z2, w1)
    s = _sc_prop(u, srcp, dstp)
    outp = _tc_last(s, u, dinv2, z2, w1, woutwp, woutbp)
    return outp[:N, :NCLS]
